# baseline (speedup 1.0000x reference)
import functools

import jax
import jax.numpy as jnp
from jax import lax
from jax.experimental import pallas as pl
from jax.experimental.pallas import tpu as pltpu
from jax.experimental.pallas import tpu_sc as plsc

NUM_GENES = 100000
NUM_SPOTS = 1000000
LATENT_DIM = 32
BATCH = 16384

_NC = 2
_NS = 16
_L = 16
_NW = _NC * _NS
_BPW = BATCH // _NW


def _nmf_body(gidx_hbm, sidx_hbm, gtab_hbm, stab_hbm, out_hbm,
              gidx_v, sidx_v, grows_v, srows_v, out_v, sem_g, sem_s):
    wid = lax.axis_index("s") * _NC + lax.axis_index("c")
    base = wid * _BPW

    pltpu.sync_copy(gidx_hbm.at[pl.ds(base, _BPW)], gidx_v)
    pltpu.sync_copy(sidx_hbm.at[pl.ds(base, _BPW)], sidx_v)

    cg = pltpu.async_copy(gtab_hbm.at[gidx_v], grows_v, sem_g)
    cs = pltpu.async_copy(stab_hbm.at[sidx_v], srows_v, sem_s)
    cg.wait()
    cs.wait()

    def block(r, carry):
        rows = r * _L + lax.iota(jnp.int32, _L)
        acc = jnp.zeros((_L,), jnp.float32)
        for d in range(LATENT_DIM):
            col = jnp.full((_L,), d, jnp.int32)
            g = plsc.load_gather(grows_v, [rows, col])
            s = plsc.load_gather(srows_v, [rows, col])
            acc = acc + g * s
        out_v[pl.ds(r * _L, _L)] = acc
        return carry

    lax.fori_loop(0, _BPW // _L, block, 0)
    pltpu.sync_copy(out_v, out_hbm.at[pl.ds(base, _BPW)])


@jax.jit
def _nmf_sc(gene_indices, spot_indices, embedding_genes, embedding_spots):
    mesh = plsc.VectorSubcoreMesh(core_axis_name="c", subcore_axis_name="s")
    run = functools.partial(
        pl.kernel,
        out_type=jax.ShapeDtypeStruct((BATCH,), jnp.float32),
        mesh=mesh,
        compiler_params=pltpu.CompilerParams(
            use_tc_tiling_on_sc=False, needs_layout_passes=False),
        scratch_types=[
            pltpu.VMEM((_BPW,), jnp.int32),
            pltpu.VMEM((_BPW,), jnp.int32),
            pltpu.VMEM((_BPW, LATENT_DIM), jnp.float32),
            pltpu.VMEM((_BPW, LATENT_DIM), jnp.float32),
            pltpu.VMEM((_BPW,), jnp.float32),
            pltpu.SemaphoreType.DMA,
            pltpu.SemaphoreType.DMA,
        ],
    )(_nmf_body)
    return run(gene_indices, spot_indices, embedding_genes, embedding_spots)


def kernel(gene_indices, spot_indices, embedding_genes, embedding_spots):
    gene_indices = gene_indices.astype(jnp.int32)
    spot_indices = spot_indices.astype(jnp.int32)
    return _nmf_sc(gene_indices, spot_indices, embedding_genes,
                   embedding_spots)
